# Initial kernel scaffold; baseline (speedup 1.0000x reference)
import jax
import jax.numpy as jnp
from jax.experimental import pallas as pl


def kernel(x, edge_index, batch, W1, b1, W2, b2, Wfc, bfc):
    raise NotImplementedError("write your pallas kernel here")


# speedup vs baseline: 34.1602x; 34.1602x over previous
import functools

import jax
import jax.numpy as jnp
from jax import lax
from jax.experimental import pallas as pl
from jax.experimental.pallas import tpu as pltpu
from jax.experimental.pallas import tpu_sc as plsc

N = 10000
E = 320000
G = 64
IN_DIM = 128
HID = 64

NC = 2
NS = 16
NW = NC * NS
PER_W = E // NW
CHUNK = 1000
STEPS = PER_W // CHUNK

_mesh = plsc.VectorSubcoreMesh(core_axis_name="c", subcore_axis_name="s")
_SC_PARAMS = pltpu.CompilerParams(use_tc_tiling_on_sc=False)


def _sc_degree(dst, zeros16):
    @functools.partial(
        pl.kernel,
        mesh=_mesh,
        compiler_params=_SC_PARAMS,
        out_type=jax.ShapeDtypeStruct((NC, N, 16), jnp.float32),
        scratch_types=[
            pltpu.VMEM((CHUNK,), jnp.int32),
            pltpu.VMEM((CHUNK, 16), jnp.float32),
            pltpu.VMEM_SHARED((N, 16), jnp.float32),
        ],
    )
    def k(dst_hbm, z_hbm, out_hbm, idx_v, ones_v, acc_sh):
        cid = lax.axis_index("c")
        sid = lax.axis_index("s")
        wid = cid * NS + sid

        @pl.loop(0, CHUNK)
        def _(r):
            ones_v[r, :] = jnp.ones((16,), jnp.float32)

        @pl.when(sid == 0)
        def _():
            pltpu.sync_copy(z_hbm, acc_sh)

        plsc.subcore_barrier()
        base = wid * PER_W

        @pl.loop(0, STEPS)
        def _(it):
            off = base + it * CHUNK
            pltpu.sync_copy(dst_hbm.at[pl.ds(off, CHUNK)], idx_v)
            pltpu.sync_copy(ones_v, acc_sh.at[idx_v], add=True)

        plsc.subcore_barrier()

        @pl.when(sid == 0)
        def _():
            pltpu.sync_copy(acc_sh, out_hbm.at[cid])

    return k(dst, zeros16)


def _sc_gather_scatter(g, src, dst, zeros64):
    @functools.partial(
        pl.kernel,
        mesh=_mesh,
        compiler_params=_SC_PARAMS,
        out_type=jax.ShapeDtypeStruct((NC, N, HID), jnp.float32),
        scratch_types=[
            pltpu.VMEM((CHUNK,), jnp.int32),
            pltpu.VMEM((CHUNK,), jnp.int32),
            pltpu.VMEM((CHUNK, HID), jnp.float32),
            pltpu.VMEM_SHARED((N, HID), jnp.float32),
            pltpu.SemaphoreType.DMA,
        ],
    )
    def k(g_hbm, src_hbm, dst_hbm, z_hbm, out_hbm, src_v, dst_v, rows_v, acc_sh, sem):
        cid = lax.axis_index("c")
        sid = lax.axis_index("s")
        wid = cid * NS + sid

        @pl.when(sid == 0)
        def _():
            pltpu.sync_copy(z_hbm, acc_sh)

        plsc.subcore_barrier()
        base = wid * PER_W

        @pl.loop(0, STEPS)
        def _(it):
            off = base + it * CHUNK
            pltpu.sync_copy(src_hbm.at[pl.ds(off, CHUNK)], src_v)
            pltpu.sync_copy(dst_hbm.at[pl.ds(off, CHUNK)], dst_v)
            pltpu.async_copy(g_hbm.at[src_v], rows_v, sem).wait()
            pltpu.sync_copy(rows_v, acc_sh.at[dst_v], add=True)

        plsc.subcore_barrier()

        @pl.when(sid == 0)
        def _():
            pltpu.sync_copy(acc_sh, out_hbm.at[cid])

    return k(g, src, dst, zeros64)


def _tc_prep(degp, x, W1):
    def body(deg_ref, x_ref, w_ref, g_ref, d_ref):
        p = deg_ref[...]
        deg = (jnp.sum(p[0], axis=1) + jnp.sum(p[1], axis=1)) * (1.0 / 16.0) + 1.0
        dinv = lax.rsqrt(jnp.maximum(deg, 1.0))
        d64 = jnp.broadcast_to(dinv[:, None], (N, HID))
        d_ref[...] = d64
        g_ref[...] = d64 * jnp.dot(x_ref[...], w_ref[...],
                                   preferred_element_type=jnp.float32)

    return pl.pallas_call(
        body,
        out_shape=(jax.ShapeDtypeStruct((N, HID), jnp.float32),
                   jax.ShapeDtypeStruct((N, HID), jnp.float32)),
    )(degp, x, W1)


def _tc_mid(acc, g, d64, b, W):
    def body(a_ref, g_ref, d_ref, b_ref, w_ref, o_ref):
        d = d_ref[...]
        h = jnp.maximum(d * (a_ref[0] + a_ref[1] + g_ref[...]) + b_ref[...][None, :], 0.0)
        o_ref[...] = d * jnp.dot(h, w_ref[...], preferred_element_type=jnp.float32)

    return pl.pallas_call(
        body,
        out_shape=jax.ShapeDtypeStruct((N, HID), jnp.float32),
    )(acc, g, d64, b, W)


def _tc_final(acc, g, d64, b, batch, Wfc, bfc):
    def body(a_ref, g_ref, d_ref, b_ref, bt_ref, wfc_ref, bfc_ref, o_ref):
        d = d_ref[...]
        h = jnp.maximum(d * (a_ref[0] + a_ref[1] + g_ref[...]) + b_ref[...][None, :], 0.0)
        bt = bt_ref[...]
        mask = (bt[None, :] == lax.broadcasted_iota(jnp.int32, (G, N), 0)
                ).astype(jnp.float32)
        counts = jnp.sum(mask, axis=1)
        pooled = jnp.dot(mask, h, preferred_element_type=jnp.float32, precision=lax.Precision.HIGHEST)
        pooled = pooled / jnp.maximum(counts, 1.0)[:, None]
        o_ref[...] = jnp.dot(pooled, wfc_ref[...],
                             preferred_element_type=jnp.float32) + bfc_ref[...][None, :]

    return pl.pallas_call(
        body,
        out_shape=jax.ShapeDtypeStruct((G, 1), jnp.float32),
    )(acc, g, d64, b, batch, Wfc, bfc)


def kernel(x, edge_index, batch, W1, b1, W2, b2, Wfc, bfc):
    src = edge_index[0]
    dst = edge_index[1]
    zeros16 = jnp.zeros((N, 16), jnp.float32)
    zeros64 = jnp.zeros((N, HID), jnp.float32)

    degp = _sc_degree(dst, zeros16)
    g1, d64 = _tc_prep(degp, x, W1)
    acc1 = _sc_gather_scatter(g1, src, dst, zeros64)
    g2 = _tc_mid(acc1, g1, d64, b1, W2)
    acc2 = _sc_gather_scatter(g2, src, dst, zeros64)
    out = _tc_final(acc2, g2, d64, b2, batch, Wfc, bfc)
    return out.reshape(G)


# speedup vs baseline: 43.2426x; 1.2659x over previous
import functools

import jax
import jax.numpy as jnp
from jax import lax
from jax.experimental import pallas as pl
from jax.experimental.pallas import tpu as pltpu
from jax.experimental.pallas import tpu_sc as plsc

N = 10000
E = 320000
G = 64
IN_DIM = 128
HID = 64

NC = 2
NS = 16
NW = NC * NS
PER_W = E // NW
CHUNK = 400
STEPS = PER_W // CHUNK
ROWS_T = N // NS

_mesh = plsc.VectorSubcoreMesh(core_axis_name="c", subcore_axis_name="s")
_SC_PARAMS = pltpu.CompilerParams(use_tc_tiling_on_sc=False)


def _sc_degree(dst2d, zeros16):
    @functools.partial(
        pl.kernel,
        mesh=_mesh,
        compiler_params=_SC_PARAMS,
        out_type=jax.ShapeDtypeStruct((NC, N, 16), jnp.float32),
        scratch_types=[
            pltpu.VMEM((STEPS, CHUNK), jnp.int32),
            pltpu.VMEM((CHUNK, 16), jnp.float32),
            pltpu.VMEM_SHARED((N, 16), jnp.float32),
            pltpu.SemaphoreType.DMA,
        ],
    )
    def k(dst_hbm, z_hbm, out_hbm, idx_v, ones_v, acc_sh, sem):
        cid = lax.axis_index("c")
        sid = lax.axis_index("s")
        wid = cid * NS + sid
        r0 = sid * ROWS_T

        @pl.loop(0, CHUNK)
        def _(r):
            ones_v[r, :] = jnp.ones((16,), jnp.float32)

        pltpu.sync_copy(z_hbm.at[pl.ds(r0, ROWS_T)], acc_sh.at[pl.ds(r0, ROWS_T)])
        pltpu.sync_copy(dst_hbm.at[pl.ds(wid * STEPS, STEPS)], idx_v)
        plsc.subcore_barrier()

        handles = [
            pltpu.async_copy(ones_v, acc_sh.at[idx_v.at[i]], sem, add=True)
            for i in range(STEPS)
        ]
        for h in handles:
            h.wait()

        plsc.subcore_barrier()
        pltpu.sync_copy(acc_sh.at[pl.ds(r0, ROWS_T)],
                        out_hbm.at[cid].at[pl.ds(r0, ROWS_T)])

    return k(dst2d, zeros16)


def _sc_gather_scatter(g, src2d, dst2d, zeros64):
    @functools.partial(
        pl.kernel,
        mesh=_mesh,
        compiler_params=_SC_PARAMS,
        out_type=jax.ShapeDtypeStruct((NC, N, HID), jnp.float32),
        scratch_types=[
            pltpu.VMEM((STEPS, CHUNK), jnp.int32),
            pltpu.VMEM((STEPS, CHUNK), jnp.int32),
            pltpu.VMEM((CHUNK, HID), jnp.float32),
            pltpu.VMEM((CHUNK, HID), jnp.float32),
            pltpu.VMEM_SHARED((N, HID), jnp.float32),
            pltpu.SemaphoreType.DMA,
            pltpu.SemaphoreType.DMA,
        ],
    )
    def k(g_hbm, src_hbm, dst_hbm, z_hbm, out_hbm,
          src_v, dst_v, rows0, rows1, acc_sh, sem_g, sem_s):
        cid = lax.axis_index("c")
        sid = lax.axis_index("s")
        wid = cid * NS + sid
        r0 = sid * ROWS_T

        pltpu.sync_copy(z_hbm.at[pl.ds(r0, ROWS_T)], acc_sh.at[pl.ds(r0, ROWS_T)])
        pltpu.sync_copy(src_hbm.at[pl.ds(wid * STEPS, STEPS)], src_v)
        pltpu.sync_copy(dst_hbm.at[pl.ds(wid * STEPS, STEPS)], dst_v)
        plsc.subcore_barrier()

        bufs = (rows0, rows1)
        g_h = [None] * STEPS
        s_h = [None] * STEPS
        g_h[0] = pltpu.async_copy(g_hbm.at[src_v.at[0]], bufs[0], sem_g)
        for i in range(STEPS):
            if i + 1 < STEPS:
                if i >= 1:
                    s_h[i - 1].wait()
                g_h[i + 1] = pltpu.async_copy(
                    g_hbm.at[src_v.at[i + 1]], bufs[(i + 1) % 2], sem_g)
            g_h[i].wait()
            s_h[i] = pltpu.async_copy(
                bufs[i % 2], acc_sh.at[dst_v.at[i]], sem_s, add=True)
        s_h[STEPS - 2].wait()
        s_h[STEPS - 1].wait()

        plsc.subcore_barrier()
        pltpu.sync_copy(acc_sh.at[pl.ds(r0, ROWS_T)],
                        out_hbm.at[cid].at[pl.ds(r0, ROWS_T)])

    return k(g, src2d, dst2d, zeros64)


def _tc_prep(degp, x, W1):
    def body(deg_ref, x_ref, w_ref, g_ref, d_ref):
        p = deg_ref[...]
        deg = (jnp.sum(p[0], axis=1) + jnp.sum(p[1], axis=1)) * (1.0 / 16.0) + 1.0
        dinv = lax.rsqrt(jnp.maximum(deg, 1.0))
        d64 = jnp.broadcast_to(dinv[:, None], (N, HID))
        d_ref[...] = d64
        g_ref[...] = d64 * jnp.dot(x_ref[...], w_ref[...],
                                   preferred_element_type=jnp.float32)

    return pl.pallas_call(
        body,
        out_shape=(jax.ShapeDtypeStruct((N, HID), jnp.float32),
                   jax.ShapeDtypeStruct((N, HID), jnp.float32)),
    )(degp, x, W1)


def _tc_mid(acc, g, d64, b, W):
    def body(a_ref, g_ref, d_ref, b_ref, w_ref, o_ref):
        d = d_ref[...]
        h = jnp.maximum(d * (a_ref[0] + a_ref[1] + g_ref[...]) + b_ref[...][None, :], 0.0)
        o_ref[...] = d * jnp.dot(h, w_ref[...], preferred_element_type=jnp.float32)

    return pl.pallas_call(
        body,
        out_shape=jax.ShapeDtypeStruct((N, HID), jnp.float32),
    )(acc, g, d64, b, W)


def _tc_final(acc, g, d64, b, batch, Wfc, bfc):
    def body(a_ref, g_ref, d_ref, b_ref, bt_ref, wfc_ref, bfc_ref, o_ref):
        d = d_ref[...]
        h = jnp.maximum(d * (a_ref[0] + a_ref[1] + g_ref[...]) + b_ref[...][None, :], 0.0)
        bt = bt_ref[...]
        mask = (bt[None, :] == lax.broadcasted_iota(jnp.int32, (G, N), 0)
                ).astype(jnp.float32)
        counts = jnp.sum(mask, axis=1)
        pooled = jnp.dot(mask, h, preferred_element_type=jnp.float32, precision=lax.Precision.HIGHEST)
        pooled = pooled / jnp.maximum(counts, 1.0)[:, None]
        o_ref[...] = jnp.dot(pooled, wfc_ref[...],
                             preferred_element_type=jnp.float32) + bfc_ref[...][None, :]

    return pl.pallas_call(
        body,
        out_shape=jax.ShapeDtypeStruct((G, 1), jnp.float32),
    )(acc, g, d64, b, batch, Wfc, bfc)


def kernel(x, edge_index, batch, W1, b1, W2, b2, Wfc, bfc):
    src2d = edge_index[0].reshape(E // CHUNK, CHUNK)
    dst2d = edge_index[1].reshape(E // CHUNK, CHUNK)
    zeros16 = jnp.zeros((N, 16), jnp.float32)
    zeros64 = jnp.zeros((N, HID), jnp.float32)

    degp = _sc_degree(dst2d, zeros16)
    g1, d64 = _tc_prep(degp, x, W1)
    acc1 = _sc_gather_scatter(g1, src2d, dst2d, zeros64)
    g2 = _tc_mid(acc1, g1, d64, b1, W2)
    acc2 = _sc_gather_scatter(g2, src2d, dst2d, zeros64)
    out = _tc_final(acc2, g2, d64, b2, batch, Wfc, bfc)
    return out.reshape(G)


# speedup vs baseline: 46.0978x; 1.0660x over previous
import functools

import jax
import jax.numpy as jnp
from jax import lax
from jax.experimental import pallas as pl
from jax.experimental.pallas import tpu as pltpu
from jax.experimental.pallas import tpu_sc as plsc

N = 10000
E = 320000
G = 64
IN_DIM = 128
HID = 64

NC = 2
NS = 16
NW = NC * NS
PER_W = E // NW
CHUNK = 400
STEPS = PER_W // CHUNK
ROWS_T = N // NS

_mesh = plsc.VectorSubcoreMesh(core_axis_name="c", subcore_axis_name="s")
_SC_PARAMS = pltpu.CompilerParams(use_tc_tiling_on_sc=False)


def _sc_degree(dst2d, zeros16):
    @functools.partial(
        pl.kernel,
        mesh=_mesh,
        compiler_params=_SC_PARAMS,
        out_type=jax.ShapeDtypeStruct((NC, N, 16), jnp.float32),
        scratch_types=[
            pltpu.VMEM((STEPS, CHUNK), jnp.int32),
            pltpu.VMEM((CHUNK, 16), jnp.float32),
            pltpu.VMEM_SHARED((N, 16), jnp.float32),
            pltpu.SemaphoreType.DMA,
        ],
    )
    def k(dst_hbm, z_hbm, out_hbm, idx_v, ones_v, acc_sh, sem):
        cid = lax.axis_index("c")
        sid = lax.axis_index("s")
        wid = cid * NS + sid
        r0 = sid * ROWS_T

        @pl.loop(0, CHUNK)
        def _(r):
            ones_v[r, :] = jnp.ones((16,), jnp.float32)

        pltpu.sync_copy(z_hbm.at[pl.ds(r0, ROWS_T)], acc_sh.at[pl.ds(r0, ROWS_T)])
        pltpu.sync_copy(dst_hbm.at[pl.ds(wid * STEPS, STEPS)], idx_v)
        plsc.subcore_barrier()

        handles = [
            pltpu.async_copy(ones_v, acc_sh.at[idx_v.at[i]], sem, add=True)
            for i in range(STEPS)
        ]
        for h in handles:
            h.wait()

        plsc.subcore_barrier()
        pltpu.sync_copy(acc_sh.at[pl.ds(r0, ROWS_T)],
                        out_hbm.at[cid].at[pl.ds(r0, ROWS_T)])

    return k(dst2d, zeros16)


def _sc_gather_scatter(g, src2d, dst2d, zeros64):
    @functools.partial(
        pl.kernel,
        mesh=_mesh,
        compiler_params=_SC_PARAMS,
        out_type=jax.ShapeDtypeStruct((NC, N, HID), jnp.float32),
        scratch_types=[
            pltpu.VMEM((STEPS, CHUNK), jnp.int32),
            pltpu.VMEM((STEPS, CHUNK), jnp.int32),
            pltpu.VMEM((CHUNK, HID), jnp.float32),
            pltpu.VMEM((CHUNK, HID), jnp.float32),
            pltpu.VMEM_SHARED((N, HID), jnp.float32),
            pltpu.SemaphoreType.DMA,
            pltpu.SemaphoreType.DMA,
        ],
    )
    def k(g_hbm, src_hbm, dst_hbm, z_hbm, out_hbm,
          src_v, dst_v, rows0, rows1, acc_sh, sem_g, sem_s):
        cid = lax.axis_index("c")
        sid = lax.axis_index("s")
        wid = cid * NS + sid
        r0 = sid * ROWS_T

        pltpu.sync_copy(z_hbm.at[pl.ds(r0, ROWS_T)], acc_sh.at[pl.ds(r0, ROWS_T)])
        pltpu.sync_copy(src_hbm.at[pl.ds(wid * STEPS, STEPS)], src_v)
        pltpu.sync_copy(dst_hbm.at[pl.ds(wid * STEPS, STEPS)], dst_v)
        plsc.subcore_barrier()

        bufs = (rows0, rows1)
        g_h = [None] * STEPS
        s_h = [None] * STEPS
        g_h[0] = pltpu.async_copy(g_hbm.at[src_v.at[0]], bufs[0], sem_g)
        for i in range(STEPS):
            if i + 1 < STEPS:
                if i >= 1:
                    s_h[i - 1].wait()
                g_h[i + 1] = pltpu.async_copy(
                    g_hbm.at[src_v.at[i + 1]], bufs[(i + 1) % 2], sem_g)
            g_h[i].wait()
            s_h[i] = pltpu.async_copy(
                bufs[i % 2], acc_sh.at[dst_v.at[i]], sem_s, add=True)
        s_h[STEPS - 2].wait()
        s_h[STEPS - 1].wait()

        plsc.subcore_barrier()
        pltpu.sync_copy(acc_sh.at[pl.ds(r0, ROWS_T)],
                        out_hbm.at[cid].at[pl.ds(r0, ROWS_T)])

    return k(g, src2d, dst2d, zeros64)


def _tc_split(edge_index):
    def body(e_ref, s_ref, d_ref):
        s_ref[...] = e_ref[0]
        d_ref[...] = e_ref[1]

    return pl.pallas_call(
        body,
        out_shape=(jax.ShapeDtypeStruct((E,), jnp.int32),
                   jax.ShapeDtypeStruct((E,), jnp.int32)),
    )(edge_index)


def _tc_prep(degp, x, W1):
    def body(deg_ref, x_ref, w_ref, g_ref, d_ref):
        p = deg_ref[...]
        deg = (jnp.sum(p[0], axis=1) + jnp.sum(p[1], axis=1)) * (1.0 / 16.0) + 1.0
        dinv = lax.rsqrt(jnp.maximum(deg, 1.0))
        d64 = jnp.broadcast_to(dinv[:, None], (N, HID))
        d_ref[...] = d64
        g_ref[...] = d64 * jnp.dot(x_ref[...], w_ref[...],
                                   preferred_element_type=jnp.float32)

    return pl.pallas_call(
        body,
        out_shape=(jax.ShapeDtypeStruct((N, HID), jnp.float32),
                   jax.ShapeDtypeStruct((N, HID), jnp.float32)),
    )(degp, x, W1)


def _tc_mid(acc, g, d64, b, W):
    def body(a_ref, g_ref, d_ref, b_ref, w_ref, o_ref):
        d = d_ref[...]
        h = jnp.maximum(d * (a_ref[0] + a_ref[1] + g_ref[...]) + b_ref[...][None, :], 0.0)
        o_ref[...] = d * jnp.dot(h, w_ref[...], preferred_element_type=jnp.float32)

    return pl.pallas_call(
        body,
        out_shape=jax.ShapeDtypeStruct((N, HID), jnp.float32),
    )(acc, g, d64, b, W)


def _tc_final(acc, g, d64, b, batch, Wfc, bfc):
    def body(a_ref, g_ref, d_ref, b_ref, bt_ref, wfc_ref, bfc_ref, o_ref):
        d = d_ref[...]
        h = jnp.maximum(d * (a_ref[0] + a_ref[1] + g_ref[...]) + b_ref[...][None, :], 0.0)
        bt = bt_ref[...]
        mask = (bt[None, :] == lax.broadcasted_iota(jnp.int32, (G, N), 0)
                ).astype(jnp.float32)
        counts = jnp.sum(mask, axis=1)
        pooled = jnp.dot(mask, h, preferred_element_type=jnp.float32, precision=lax.Precision.HIGHEST)
        pooled = pooled / jnp.maximum(counts, 1.0)[:, None]
        o_ref[...] = jnp.dot(pooled, wfc_ref[...],
                             preferred_element_type=jnp.float32) + bfc_ref[...][None, :]

    return pl.pallas_call(
        body,
        out_shape=jax.ShapeDtypeStruct((G, 1), jnp.float32),
    )(acc, g, d64, b, batch, Wfc, bfc)


def kernel(x, edge_index, batch, W1, b1, W2, b2, Wfc, bfc):
    src1d, dst1d = _tc_split(edge_index)
    src2d = src1d.reshape(E // CHUNK, CHUNK)
    dst2d = dst1d.reshape(E // CHUNK, CHUNK)
    zeros16 = jnp.zeros((N, 16), jnp.float32)
    zeros64 = jnp.zeros((N, HID), jnp.float32)

    degp = _sc_degree(dst2d, zeros16)
    g1, d64 = _tc_prep(degp, x, W1)
    acc1 = _sc_gather_scatter(g1, src2d, dst2d, zeros64)
    g2 = _tc_mid(acc1, g1, d64, b1, W2)
    acc2 = _sc_gather_scatter(g2, src2d, dst2d, zeros64)
    out = _tc_final(acc2, g2, d64, b2, batch, Wfc, bfc)
    return out.reshape(G)


# speedup vs baseline: 47.8052x; 1.0370x over previous
import functools

import jax
import jax.numpy as jnp
from jax import lax
from jax.experimental import pallas as pl
from jax.experimental.pallas import tpu as pltpu
from jax.experimental.pallas import tpu_sc as plsc

N = 10000
E = 320000
G = 64
IN_DIM = 128
HID = 64

NC = 2
NS = 16
NW = NC * NS
PER_W = E // NW
CHUNK = 200
STEPS = PER_W // CHUNK
NBUF = 4
ROWS_T = N // NS

_mesh = plsc.VectorSubcoreMesh(core_axis_name="c", subcore_axis_name="s")
_SC_PARAMS = pltpu.CompilerParams(use_tc_tiling_on_sc=False)


def _sc_degree(dst2d, zeros16):
    @functools.partial(
        pl.kernel,
        mesh=_mesh,
        compiler_params=_SC_PARAMS,
        out_type=jax.ShapeDtypeStruct((NC, N, 16), jnp.float32),
        scratch_types=[
            pltpu.VMEM((STEPS, CHUNK), jnp.int32),
            pltpu.VMEM((CHUNK, 16), jnp.float32),
            pltpu.VMEM_SHARED((N, 16), jnp.float32),
            pltpu.SemaphoreType.DMA,
        ],
    )
    def k(dst_hbm, z_hbm, out_hbm, idx_v, ones_v, acc_sh, sem):
        cid = lax.axis_index("c")
        sid = lax.axis_index("s")
        wid = cid * NS + sid
        r0 = sid * ROWS_T

        @pl.loop(0, CHUNK)
        def _(r):
            ones_v[r, :] = jnp.ones((16,), jnp.float32)

        pltpu.sync_copy(z_hbm.at[pl.ds(r0, ROWS_T)], acc_sh.at[pl.ds(r0, ROWS_T)])
        pltpu.sync_copy(dst_hbm.at[pl.ds(wid * STEPS, STEPS)], idx_v)
        plsc.subcore_barrier()

        handles = [
            pltpu.async_copy(ones_v, acc_sh.at[idx_v.at[i]], sem, add=True)
            for i in range(STEPS)
        ]
        for h in handles:
            h.wait()

        plsc.subcore_barrier()
        pltpu.sync_copy(acc_sh.at[pl.ds(r0, ROWS_T)],
                        out_hbm.at[cid].at[pl.ds(r0, ROWS_T)])

    return k(dst2d, zeros16)


def _sc_gather_scatter(g, src2d, dst2d, zeros64):
    @functools.partial(
        pl.kernel,
        mesh=_mesh,
        compiler_params=_SC_PARAMS,
        out_type=jax.ShapeDtypeStruct((NC, N, HID), jnp.float32),
        scratch_types=[
            pltpu.VMEM((STEPS, CHUNK), jnp.int32),
            pltpu.VMEM((STEPS, CHUNK), jnp.int32),
        ] + [pltpu.VMEM((CHUNK, HID), jnp.float32) for _ in range(NBUF)] + [
            pltpu.VMEM_SHARED((N, HID), jnp.float32),
            pltpu.SemaphoreType.DMA,
            pltpu.SemaphoreType.DMA,
        ],
    )
    def k(g_hbm, src_hbm, dst_hbm, z_hbm, out_hbm,
          src_v, dst_v, *rest):
        bufs = rest[:NBUF]
        acc_sh, sem_g, sem_s = rest[NBUF:]
        cid = lax.axis_index("c")
        sid = lax.axis_index("s")
        wid = cid * NS + sid
        r0 = sid * ROWS_T

        pltpu.sync_copy(z_hbm.at[pl.ds(r0, ROWS_T)], acc_sh.at[pl.ds(r0, ROWS_T)])
        pltpu.sync_copy(src_hbm.at[pl.ds(wid * STEPS, STEPS)], src_v)
        pltpu.sync_copy(dst_hbm.at[pl.ds(wid * STEPS, STEPS)], dst_v)
        plsc.subcore_barrier()

        g_h = [None] * STEPS
        s_h = [None] * STEPS
        for j in range(min(NBUF - 1, STEPS)):
            g_h[j] = pltpu.async_copy(g_hbm.at[src_v.at[j]], bufs[j % NBUF], sem_g)
        for i in range(STEPS):
            if i + NBUF - 1 < STEPS:
                if i >= 1:
                    s_h[i - 1].wait()
                g_h[i + NBUF - 1] = pltpu.async_copy(
                    g_hbm.at[src_v.at[i + NBUF - 1]],
                    bufs[(i + NBUF - 1) % NBUF], sem_g)
            g_h[i].wait()
            s_h[i] = pltpu.async_copy(
                bufs[i % NBUF], acc_sh.at[dst_v.at[i]], sem_s, add=True)
        for i in range(max(0, STEPS - NBUF), STEPS):
            s_h[i].wait()

        plsc.subcore_barrier()
        pltpu.sync_copy(acc_sh.at[pl.ds(r0, ROWS_T)],
                        out_hbm.at[cid].at[pl.ds(r0, ROWS_T)])

    return k(g, src2d, dst2d, zeros64)


def _tc_split(edge_index):
    def body(e_ref, s_ref, d_ref):
        s_ref[...] = e_ref[0]
        d_ref[...] = e_ref[1]

    return pl.pallas_call(
        body,
        out_shape=(jax.ShapeDtypeStruct((E,), jnp.int32),
                   jax.ShapeDtypeStruct((E,), jnp.int32)),
    )(edge_index)


def _tc_prep(degp, x, W1):
    def body(deg_ref, x_ref, w_ref, g_ref, d_ref):
        p = deg_ref[...]
        deg = (jnp.sum(p[0], axis=1) + jnp.sum(p[1], axis=1)) * (1.0 / 16.0) + 1.0
        dinv = lax.rsqrt(jnp.maximum(deg, 1.0))
        d64 = jnp.broadcast_to(dinv[:, None], (N, HID))
        d_ref[...] = d64
        g_ref[...] = d64 * jnp.dot(x_ref[...], w_ref[...],
                                   preferred_element_type=jnp.float32)

    return pl.pallas_call(
        body,
        out_shape=(jax.ShapeDtypeStruct((N, HID), jnp.float32),
                   jax.ShapeDtypeStruct((N, HID), jnp.float32)),
    )(degp, x, W1)


def _tc_mid(acc, g, d64, b, W):
    def body(a_ref, g_ref, d_ref, b_ref, w_ref, o_ref):
        d = d_ref[...]
        h = jnp.maximum(d * (a_ref[0] + a_ref[1] + g_ref[...]) + b_ref[...][None, :], 0.0)
        o_ref[...] = d * jnp.dot(h, w_ref[...], preferred_element_type=jnp.float32)

    return pl.pallas_call(
        body,
        out_shape=jax.ShapeDtypeStruct((N, HID), jnp.float32),
    )(acc, g, d64, b, W)


def _tc_final(acc, g, d64, b, batch, Wfc, bfc):
    def body(a_ref, g_ref, d_ref, b_ref, bt_ref, wfc_ref, bfc_ref, o_ref):
        d = d_ref[...]
        h = jnp.maximum(d * (a_ref[0] + a_ref[1] + g_ref[...]) + b_ref[...][None, :], 0.0)
        bt = bt_ref[...]
        mask = (bt[None, :] == lax.broadcasted_iota(jnp.int32, (G, N), 0)
                ).astype(jnp.float32)
        counts = jnp.sum(mask, axis=1)
        pooled = jnp.dot(mask, h, preferred_element_type=jnp.float32, precision=lax.Precision.HIGHEST)
        pooled = pooled / jnp.maximum(counts, 1.0)[:, None]
        o_ref[...] = jnp.dot(pooled, wfc_ref[...],
                             preferred_element_type=jnp.float32) + bfc_ref[...][None, :]

    return pl.pallas_call(
        body,
        out_shape=jax.ShapeDtypeStruct((G, 1), jnp.float32),
    )(acc, g, d64, b, batch, Wfc, bfc)


def kernel(x, edge_index, batch, W1, b1, W2, b2, Wfc, bfc):
    src1d, dst1d = _tc_split(edge_index)
    src2d = src1d.reshape(E // CHUNK, CHUNK)
    dst2d = dst1d.reshape(E // CHUNK, CHUNK)
    zeros16 = jnp.zeros((N, 16), jnp.float32)
    zeros64 = jnp.zeros((N, HID), jnp.float32)

    degp = _sc_degree(dst2d, zeros16)
    g1, d64 = _tc_prep(degp, x, W1)
    acc1 = _sc_gather_scatter(g1, src2d, dst2d, zeros64)
    g2 = _tc_mid(acc1, g1, d64, b1, W2)
    acc2 = _sc_gather_scatter(g2, src2d, dst2d, zeros64)
    out = _tc_final(acc2, g2, d64, b2, batch, Wfc, bfc)
    return out.reshape(G)


# speedup vs baseline: 58.3255x; 1.2201x over previous
import functools

import jax
import jax.numpy as jnp
from jax import lax
from jax.experimental import pallas as pl
from jax.experimental.pallas import tpu as pltpu
from jax.experimental.pallas import tpu_sc as plsc

N = 10000
E = 320000
G = 64
IN_DIM = 128
HID = 64

NC = 2
NS = 16
NW = NC * NS
PER_W = E // NW
CHUNK = 200
STEPS = PER_W // CHUNK
NBUF = 4
ROWS_T = N // NS

_mesh = plsc.VectorSubcoreMesh(core_axis_name="c", subcore_axis_name="s")
_SC_PARAMS = pltpu.CompilerParams(use_tc_tiling_on_sc=False)


def _sc_degree(dst2d, zeros16):
    @functools.partial(
        pl.kernel,
        mesh=_mesh,
        compiler_params=_SC_PARAMS,
        out_type=jax.ShapeDtypeStruct((NC, N, 16), jnp.float32),
        scratch_types=[
            pltpu.VMEM((STEPS, CHUNK), jnp.int32),
            pltpu.VMEM((CHUNK, 16), jnp.float32),
            pltpu.VMEM_SHARED((N, 16), jnp.float32),
            pltpu.SemaphoreType.DMA,
        ],
    )
    def k(dst_hbm, z_hbm, out_hbm, idx_v, ones_v, acc_sh, sem):
        cid = lax.axis_index("c")
        sid = lax.axis_index("s")
        wid = cid * NS + sid
        r0 = sid * ROWS_T

        @pl.loop(0, CHUNK)
        def _(r):
            ones_v[r, :] = jnp.ones((16,), jnp.float32)

        pltpu.sync_copy(z_hbm.at[pl.ds(r0, ROWS_T)], acc_sh.at[pl.ds(r0, ROWS_T)])
        pltpu.sync_copy(dst_hbm.at[pl.ds(wid * STEPS, STEPS)], idx_v)
        plsc.subcore_barrier()

        handles = [
            pltpu.async_copy(ones_v, acc_sh.at[idx_v.at[i]], sem, add=True)
            for i in range(STEPS)
        ]
        for h in handles:
            h.wait()

        plsc.subcore_barrier()
        pltpu.sync_copy(acc_sh.at[pl.ds(r0, ROWS_T)],
                        out_hbm.at[cid].at[pl.ds(r0, ROWS_T)])

    return k(dst2d, zeros16)


def _sc_gather_scatter(g, src2d, dst2d, zeros64):
    @functools.partial(
        pl.kernel,
        mesh=_mesh,
        compiler_params=_SC_PARAMS,
        out_type=jax.ShapeDtypeStruct((NC, N, HID), jnp.float32),
        scratch_types=[
            pltpu.VMEM((STEPS, CHUNK), jnp.int32),
            pltpu.VMEM((STEPS, CHUNK), jnp.int32),
        ] + [pltpu.VMEM((CHUNK, HID), jnp.float32) for _ in range(NBUF)] + [
            pltpu.VMEM_SHARED((N, HID), jnp.float32),
            pltpu.SemaphoreType.DMA,
            pltpu.SemaphoreType.DMA,
        ],
    )
    def k(g_hbm, src_hbm, dst_hbm, z_hbm, out_hbm,
          src_v, dst_v, *rest):
        bufs = rest[:NBUF]
        acc_sh, sem_g, sem_s = rest[NBUF:]
        cid = lax.axis_index("c")
        sid = lax.axis_index("s")
        wid = cid * NS + sid
        r0 = sid * ROWS_T

        pltpu.sync_copy(z_hbm.at[pl.ds(r0, ROWS_T)], acc_sh.at[pl.ds(r0, ROWS_T)])
        pltpu.sync_copy(src_hbm.at[pl.ds(wid * STEPS, STEPS)], src_v)
        pltpu.sync_copy(dst_hbm.at[pl.ds(wid * STEPS, STEPS)], dst_v)
        plsc.subcore_barrier()

        g_h = [None] * STEPS
        s_h = [None] * STEPS
        for j in range(min(NBUF - 1, STEPS)):
            g_h[j] = pltpu.async_copy(g_hbm.at[src_v.at[j]], bufs[j % NBUF], sem_g)
        for i in range(STEPS):
            if i + NBUF - 1 < STEPS:
                if i >= 1:
                    s_h[i - 1].wait()
                g_h[i + NBUF - 1] = pltpu.async_copy(
                    g_hbm.at[src_v.at[i + NBUF - 1]],
                    bufs[(i + NBUF - 1) % NBUF], sem_g)
            g_h[i].wait()
            s_h[i] = pltpu.async_copy(
                bufs[i % NBUF], acc_sh.at[dst_v.at[i]], sem_s, add=True)
        for i in range(max(0, STEPS - NBUF), STEPS):
            s_h[i].wait()

        plsc.subcore_barrier()
        pltpu.sync_copy(acc_sh.at[pl.ds(r0, ROWS_T)],
                        out_hbm.at[cid].at[pl.ds(r0, ROWS_T)])

    return k(g, src2d, dst2d, zeros64)


def _tc_split(edge_index):
    def body(e_ref, s_ref, d_ref):
        s_ref[...] = e_ref[0]
        d_ref[...] = e_ref[1]

    return pl.pallas_call(
        body,
        out_shape=(jax.ShapeDtypeStruct((E,), jnp.int32),
                   jax.ShapeDtypeStruct((E,), jnp.int32)),
    )(edge_index)


def _tc_prep(degF, xF, W1bd):
    def body(deg_ref, x_ref, w_ref, g_ref, d_ref):
        p = deg_ref[...]
        psum = p[0] + p[1]
        li = lax.broadcasted_iota(jnp.int32, (32, 128), 0)
        fi = lax.broadcasted_iota(jnp.int32, (32, 128), 1)
        m = ((li // 16) == (fi // HID)).astype(jnp.float32)
        t = jnp.dot(psum, m, preferred_element_type=jnp.float32,
                    precision=lax.Precision.HIGHEST)
        deg = t * (1.0 / 16.0) + 1.0
        dF = lax.rsqrt(jnp.maximum(deg, 1.0))
        d_ref[...] = dF
        g_ref[...] = dF * jnp.dot(x_ref[...], w_ref[...],
                                  preferred_element_type=jnp.float32)

    return pl.pallas_call(
        body,
        out_shape=(jax.ShapeDtypeStruct((N // 2, 128), jnp.float32),
                   jax.ShapeDtypeStruct((N // 2, 128), jnp.float32)),
    )(degF, xF, W1bd)


def _tc_mid(accF, gF, dF, bt, Wbd):
    def body(a_ref, g_ref, d_ref, b_ref, w_ref, o_ref):
        a = a_ref[...]
        aa = a[0:N // 2] + a[N // 2:N]
        d = d_ref[...]
        h = jnp.maximum(d * (aa + g_ref[...]) + b_ref[...][None, :], 0.0)
        o_ref[...] = d * jnp.dot(h, w_ref[...], preferred_element_type=jnp.float32)

    return pl.pallas_call(
        body,
        out_shape=jax.ShapeDtypeStruct((N // 2, 128), jnp.float32),
    )(accF, gF, dF, bt, Wbd)


def _tc_final(accF, gF, dF, bt, bE, bO, Wfc, bfc):
    def body(a_ref, g_ref, d_ref, b_ref, be_ref, bo_ref, wfc_ref, bfc_ref, o_ref):
        a = a_ref[...]
        aa = a[0:N // 2] + a[N // 2:N]
        d = d_ref[...]
        h = jnp.maximum(d * (aa + g_ref[...]) + b_ref[...][None, :], 0.0)
        iota = lax.broadcasted_iota(jnp.int32, (G, N // 2), 0)
        maskE = (be_ref[...][None, :] == iota).astype(jnp.float32)
        maskO = (bo_ref[...][None, :] == iota).astype(jnp.float32)
        hE = h[:, 0:HID]
        hO = h[:, HID:2 * HID]
        pooled = (jnp.dot(maskE, hE, preferred_element_type=jnp.float32,
                          precision=lax.Precision.HIGHEST)
                  + jnp.dot(maskO, hO, preferred_element_type=jnp.float32,
                            precision=lax.Precision.HIGHEST))
        counts = jnp.sum(maskE, axis=1) + jnp.sum(maskO, axis=1)
        pooled = pooled / jnp.maximum(counts, 1.0)[:, None]
        o_ref[...] = jnp.dot(pooled, wfc_ref[...],
                             preferred_element_type=jnp.float32) + bfc_ref[...][None, :]

    return pl.pallas_call(
        body,
        out_shape=jax.ShapeDtypeStruct((G, 1), jnp.float32),
    )(accF, gF, dF, bt, bE, bO, Wfc, bfc)


def _blockdiag2(W):
    zero = jnp.zeros_like(W)
    top = jnp.concatenate([W, zero], axis=1)
    bot = jnp.concatenate([zero, W], axis=1)
    return jnp.concatenate([top, bot], axis=0)


def kernel(x, edge_index, batch, W1, b1, W2, b2, Wfc, bfc):
    src1d, dst1d = _tc_split(edge_index)
    src2d = src1d.reshape(E // CHUNK, CHUNK)
    dst2d = dst1d.reshape(E // CHUNK, CHUNK)
    zeros16 = jnp.zeros((N, 16), jnp.float32)
    zeros64 = jnp.zeros((N, HID), jnp.float32)

    xF = x.reshape(N // 2, 2 * IN_DIM)
    W1bd = _blockdiag2(W1)
    W2bd = _blockdiag2(W2)
    b1t = jnp.concatenate([b1, b1])
    b2t = jnp.concatenate([b2, b2])
    b2d = batch.reshape(N // 2, 2)
    bE = b2d[:, 0]
    bO = b2d[:, 1]

    degp = _sc_degree(dst2d, zeros16)
    g1F, dF = _tc_prep(degp.reshape(NC, N // 2, 32), xF, W1bd)
    acc1 = _sc_gather_scatter(g1F.reshape(N, HID), src2d, dst2d, zeros64)
    g2F = _tc_mid(acc1.reshape(N, 2 * HID), g1F, dF, b1t, W2bd)
    acc2 = _sc_gather_scatter(g2F.reshape(N, HID), src2d, dst2d, zeros64)
    out = _tc_final(acc2.reshape(N, 2 * HID), g2F, dF, b2t, bE, bO, Wfc, bfc)
    return out.reshape(G)


# speedup vs baseline: 59.5634x; 1.0212x over previous
import functools

import jax
import jax.numpy as jnp
from jax import lax
from jax.experimental import pallas as pl
from jax.experimental.pallas import tpu as pltpu
from jax.experimental.pallas import tpu_sc as plsc

N = 10000
E = 320000
G = 64
IN_DIM = 128
HID = 64

NC = 2
NS = 16
NW = NC * NS
PER_W = E // NW
CHUNK = 200
STEPS = PER_W // CHUNK
NBUF = 5
ROWS_T = N // NS

_mesh = plsc.VectorSubcoreMesh(core_axis_name="c", subcore_axis_name="s")
_SC_PARAMS = pltpu.CompilerParams(use_tc_tiling_on_sc=False)


def _sc_degree(dst2d, zeros16):
    @functools.partial(
        pl.kernel,
        mesh=_mesh,
        compiler_params=_SC_PARAMS,
        out_type=jax.ShapeDtypeStruct((NC, N, 16), jnp.float32),
        scratch_types=[
            pltpu.VMEM((STEPS, CHUNK), jnp.int32),
            pltpu.VMEM((CHUNK, 16), jnp.float32),
            pltpu.VMEM_SHARED((N, 16), jnp.float32),
            pltpu.SemaphoreType.DMA,
        ],
    )
    def k(dst_hbm, z_hbm, out_hbm, idx_v, ones_v, acc_sh, sem):
        cid = lax.axis_index("c")
        sid = lax.axis_index("s")
        wid = cid * NS + sid
        r0 = sid * ROWS_T

        @pl.loop(0, CHUNK)
        def _(r):
            ones_v[r, :] = jnp.ones((16,), jnp.float32)

        pltpu.sync_copy(z_hbm.at[pl.ds(r0, ROWS_T)], acc_sh.at[pl.ds(r0, ROWS_T)])
        pltpu.sync_copy(dst_hbm.at[pl.ds(wid * STEPS, STEPS)], idx_v)
        plsc.subcore_barrier()

        handles = [
            pltpu.async_copy(ones_v, acc_sh.at[idx_v.at[i]], sem, add=True)
            for i in range(STEPS)
        ]
        for h in handles:
            h.wait()

        plsc.subcore_barrier()
        pltpu.sync_copy(acc_sh.at[pl.ds(r0, ROWS_T)],
                        out_hbm.at[cid].at[pl.ds(r0, ROWS_T)])

    return k(dst2d, zeros16)


def _sc_gather_scatter(g, src2d, dst2d, zeros64):
    @functools.partial(
        pl.kernel,
        mesh=_mesh,
        compiler_params=_SC_PARAMS,
        out_type=jax.ShapeDtypeStruct((NC, N, HID), jnp.float32),
        scratch_types=[
            pltpu.VMEM((STEPS, CHUNK), jnp.int32),
            pltpu.VMEM((STEPS, CHUNK), jnp.int32),
        ] + [pltpu.VMEM((CHUNK, HID), jnp.float32) for _ in range(NBUF)] + [
            pltpu.VMEM_SHARED((N, HID), jnp.float32),
            pltpu.SemaphoreType.DMA,
            pltpu.SemaphoreType.DMA,
        ],
    )
    def k(g_hbm, src_hbm, dst_hbm, z_hbm, out_hbm,
          src_v, dst_v, *rest):
        bufs = rest[:NBUF]
        acc_sh, sem_g, sem_s = rest[NBUF:]
        cid = lax.axis_index("c")
        sid = lax.axis_index("s")
        wid = cid * NS + sid
        r0 = sid * ROWS_T

        pltpu.sync_copy(z_hbm.at[pl.ds(r0, ROWS_T)], acc_sh.at[pl.ds(r0, ROWS_T)])
        pltpu.sync_copy(src_hbm.at[pl.ds(wid * STEPS, STEPS)], src_v)
        pltpu.sync_copy(dst_hbm.at[pl.ds(wid * STEPS, STEPS)], dst_v)
        plsc.subcore_barrier()

        g_h = [None] * STEPS
        s_h = [None] * STEPS
        for j in range(min(NBUF - 1, STEPS)):
            g_h[j] = pltpu.async_copy(g_hbm.at[src_v.at[j]], bufs[j % NBUF], sem_g)
        for i in range(STEPS):
            if i + NBUF - 1 < STEPS:
                if i >= 1:
                    s_h[i - 1].wait()
                g_h[i + NBUF - 1] = pltpu.async_copy(
                    g_hbm.at[src_v.at[i + NBUF - 1]],
                    bufs[(i + NBUF - 1) % NBUF], sem_g)
            g_h[i].wait()
            s_h[i] = pltpu.async_copy(
                bufs[i % NBUF], acc_sh.at[dst_v.at[i]], sem_s, add=True)
        for i in range(max(0, STEPS - NBUF), STEPS):
            s_h[i].wait()

        plsc.subcore_barrier()
        pltpu.sync_copy(acc_sh.at[pl.ds(r0, ROWS_T)],
                        out_hbm.at[cid].at[pl.ds(r0, ROWS_T)])

    return k(g, src2d, dst2d, zeros64)


def _tc_split(edge_index):
    def body(e_ref, s_ref, d_ref):
        s_ref[...] = e_ref[0]
        d_ref[...] = e_ref[1]

    return pl.pallas_call(
        body,
        out_shape=(jax.ShapeDtypeStruct((E,), jnp.int32),
                   jax.ShapeDtypeStruct((E,), jnp.int32)),
    )(edge_index)


def _tc_prep(degF, xF, W1bd):
    def body(deg_ref, x_ref, w_ref, g_ref, d_ref):
        p = deg_ref[...]
        psum = p[0] + p[1]
        li = lax.broadcasted_iota(jnp.int32, (32, 128), 0)
        fi = lax.broadcasted_iota(jnp.int32, (32, 128), 1)
        m = ((li // 16) == (fi // HID)).astype(jnp.float32)
        t = jnp.dot(psum, m, preferred_element_type=jnp.float32)
        deg = t * (1.0 / 16.0) + 1.0
        dF = lax.rsqrt(jnp.maximum(deg, 1.0))
        d_ref[...] = dF
        g_ref[...] = dF * jnp.dot(x_ref[...], w_ref[...],
                                  preferred_element_type=jnp.float32)

    return pl.pallas_call(
        body,
        out_shape=(jax.ShapeDtypeStruct((N // 2, 128), jnp.float32),
                   jax.ShapeDtypeStruct((N // 2, 128), jnp.float32)),
    )(degF, xF, W1bd)


def _tc_mid(accF, gF, dF, bt, Wbd):
    def body(a_ref, g_ref, d_ref, b_ref, w_ref, o_ref):
        a = a_ref[...]
        aa = a[0:N // 2] + a[N // 2:N]
        d = d_ref[...]
        h = jnp.maximum(d * (aa + g_ref[...]) + b_ref[...][None, :], 0.0)
        o_ref[...] = d * jnp.dot(h, w_ref[...], preferred_element_type=jnp.float32)

    return pl.pallas_call(
        body,
        out_shape=jax.ShapeDtypeStruct((N // 2, 128), jnp.float32),
    )(accF, gF, dF, bt, Wbd)


def _tc_final(accF, gF, dF, bt, bE, bO, Wfc, bfc):
    def body(a_ref, g_ref, d_ref, b_ref, be_ref, bo_ref, wfc_ref, bfc_ref, o_ref):
        a = a_ref[...]
        aa = a[0:N // 2] + a[N // 2:N]
        d = d_ref[...]
        h = jnp.maximum(d * (aa + g_ref[...]) + b_ref[...][None, :], 0.0)
        iota = lax.broadcasted_iota(jnp.int32, (G, N // 2), 0)
        maskE = (be_ref[...][None, :] == iota).astype(jnp.float32)
        maskO = (bo_ref[...][None, :] == iota).astype(jnp.float32)
        hE = h[:, 0:HID]
        hO = h[:, HID:2 * HID]
        pooled = (jnp.dot(maskE, hE, preferred_element_type=jnp.float32,
                          precision=lax.Precision.HIGHEST)
                  + jnp.dot(maskO, hO, preferred_element_type=jnp.float32,
                            precision=lax.Precision.HIGHEST))
        counts = jnp.sum(maskE, axis=1) + jnp.sum(maskO, axis=1)
        pooled = pooled / jnp.maximum(counts, 1.0)[:, None]
        o_ref[...] = jnp.dot(pooled, wfc_ref[...],
                             preferred_element_type=jnp.float32) + bfc_ref[...][None, :]

    return pl.pallas_call(
        body,
        out_shape=jax.ShapeDtypeStruct((G, 1), jnp.float32),
    )(accF, gF, dF, bt, bE, bO, Wfc, bfc)


def _blockdiag2(W):
    zero = jnp.zeros_like(W)
    top = jnp.concatenate([W, zero], axis=1)
    bot = jnp.concatenate([zero, W], axis=1)
    return jnp.concatenate([top, bot], axis=0)


def kernel(x, edge_index, batch, W1, b1, W2, b2, Wfc, bfc):
    src1d, dst1d = _tc_split(edge_index)
    src2d = src1d.reshape(E // CHUNK, CHUNK)
    dst2d = dst1d.reshape(E // CHUNK, CHUNK)
    zeros16 = jnp.zeros((N, 16), jnp.float32)
    zeros64 = jnp.zeros((N, HID), jnp.float32)

    xF = x.reshape(N // 2, 2 * IN_DIM)
    W1bd = _blockdiag2(W1)
    W2bd = _blockdiag2(W2)
    b1t = jnp.concatenate([b1, b1])
    b2t = jnp.concatenate([b2, b2])
    b2d = batch.reshape(N // 2, 2)
    bE = b2d[:, 0]
    bO = b2d[:, 1]

    degp = _sc_degree(dst2d, zeros16)
    g1F, dF = _tc_prep(degp.reshape(NC, N // 2, 32), xF, W1bd)
    acc1 = _sc_gather_scatter(g1F.reshape(N, HID), src2d, dst2d, zeros64)
    g2F = _tc_mid(acc1.reshape(N, 2 * HID), g1F, dF, b1t, W2bd)
    acc2 = _sc_gather_scatter(g2F.reshape(N, HID), src2d, dst2d, zeros64)
    out = _tc_final(acc2.reshape(N, 2 * HID), g2F, dF, b2t, bE, bO, Wfc, bfc)
    return out.reshape(G)
